# baseline (speedup 1.0000x reference)
import functools

import jax
import jax.numpy as jnp
from jax import lax
from jax.experimental import pallas as pl
from jax.experimental.pallas import tpu as pltpu
from jax.experimental.pallas import tpu_sc as plsc

N = 10000
NP = 10240
E = 320000
D = 128

NC = 2
NS = 16
NW = NC * NS
ECH = E // NW
B = 100
NB = ECH // B
ROWS_S = NP // NS

_mesh = plsc.VectorSubcoreMesh(core_axis_name="c", subcore_axis_name="s")
_sc_params = pltpu.CompilerParams(needs_layout_passes=False)


@functools.partial(
    pl.kernel,
    out_type=jax.ShapeDtypeStruct((NW, NP), jnp.float32),
    mesh=_mesh,
    scratch_types=[
        pltpu.VMEM((NP,), jnp.float32),
        pltpu.VMEM((ECH,), jnp.int32),
        pltpu.VMEM((ECH,), jnp.float32),
    ],
    compiler_params=_sc_params,
)
def _sc_deg(col_hbm, ew_hbm, z_hbm, out_hbm, deg_v, col_v, ew_v):
    c = lax.axis_index("c")
    s = lax.axis_index("s")
    w = c * NS + s
    pltpu.sync_copy(z_hbm, deg_v)
    pltpu.sync_copy(col_hbm.at[w], col_v)
    pltpu.sync_copy(ew_hbm.at[w], ew_v)

    @pl.loop(0, ECH, step=16)
    def _(i):
        plsc.addupdate_scatter(deg_v, [col_v[pl.ds(i, 16)]], ew_v[pl.ds(i, 16)])

    pltpu.sync_copy(deg_v, out_hbm.at[w])


@functools.partial(
    pl.kernel,
    out_type=jax.ShapeDtypeStruct((NC, NP, D), jnp.float32),
    mesh=_mesh,
    scratch_types=[
        pltpu.VMEM((NB, B), jnp.int32),
        pltpu.VMEM((1, B), jnp.int32),
        pltpu.VMEM((1, B), jnp.int32),
        pltpu.VMEM((B,), jnp.float32),
        pltpu.VMEM((B,), jnp.float32),
        pltpu.VMEM((B, D), jnp.float32),
        pltpu.VMEM((B, D), jnp.float32),
        pltpu.SemaphoreType.DMA,
        pltpu.SemaphoreType.DMA,
        pltpu.SemaphoreType.DMA,
        pltpu.SemaphoreType.DMA,
        pltpu.SemaphoreType.DMA,
        pltpu.SemaphoreType.DMA,
        pltpu.VMEM_SHARED((NP, D), jnp.float32),
    ],
    compiler_params=_sc_params,
)
def _sc_agg(y_hbm, row_hbm, col_hbm, ew_hbm, z_hbm, out_hbm,
            row_v, colb0, colb1, ewb0, ewb1, rows_v0, rows_v1,
            csem0, csem1, esem0, esem1, gsem0, gsem1, acc_sh):
    c = lax.axis_index("c")
    s = lax.axis_index("s")
    w = c * NS + s
    rows = (rows_v0, rows_v1)
    colb = (colb0, colb1)
    ewb = (ewb0, ewb1)
    csem = (csem0, csem1)
    esem = (esem0, esem1)
    gsem = (gsem0, gsem1)
    pltpu.sync_copy(z_hbm.at[pl.ds(s * ROWS_S, ROWS_S)],
                    acc_sh.at[pl.ds(s * ROWS_S, ROWS_S)])
    pltpu.sync_copy(row_hbm.at[w], row_v)
    plsc.subcore_barrier()

    def scale(b):
        @plsc.parallel_loop(0, B, step=1, unroll=2)
        def _(e):
            b16 = plsc.load_gather(ewb[b], [jnp.full((16,), e, jnp.int32)])
            for k in range(D // 16):
                rows[b][e, pl.ds(k * 16, 16)] = (
                    rows[b][e, pl.ds(k * 16, 16)] * b16)

    def body(j, b, nxt=True):
        nb = b ^ 1
        if nxt:
            pltpu.async_copy(y_hbm.at[row_v.at[j + 1]], rows[nb], gsem[nb])
            pltpu.async_copy(col_hbm.at[w, pl.ds(j + 1, 1)], colb[nb],
                             csem[nb])
            pltpu.async_copy(ew_hbm.at[w, j + 1], ewb[nb], esem[nb])
        pltpu.make_async_copy(y_hbm.at[row_v.at[j]], rows[b],
                              gsem[b]).wait()
        pltpu.make_async_copy(ew_hbm.at[w, j], ewb[b], esem[b]).wait()
        scale(b)
        pltpu.make_async_copy(col_hbm.at[w, pl.ds(j, 1)], colb[b],
                              csem[b]).wait()
        pltpu.sync_copy(rows[b], acc_sh.at[colb[b].at[0]], add=True)

    pltpu.async_copy(y_hbm.at[row_v.at[0]], rows[0], gsem[0])
    pltpu.async_copy(col_hbm.at[w, pl.ds(0, 1)], colb[0], csem[0])
    pltpu.async_copy(ew_hbm.at[w, 0], ewb[0], esem[0])

    @pl.loop(0, NB - 2, step=2)
    def _(j):
        body(j, 0)
        body(j + 1, 1)

    body(NB - 2, 0)
    body(NB - 1, 1, nxt=False)

    plsc.subcore_barrier()
    pltpu.sync_copy(acc_sh.at[pl.ds(s * ROWS_S, ROWS_S)],
                    out_hbm.at[c, pl.ds(s * ROWS_S, ROWS_S)])


def _tc1_body(degp_ref, x_ref, w1_ref, dis_ref, y1_ref):
    deg = jnp.sum(degp_ref[:, :N], axis=0) + 1.0
    dis = lax.rsqrt(deg)
    dis_ref[...] = dis
    xw = lax.dot_general(x_ref[...], w1_ref[...], (((1,), (1,)), ((), ())),
                         preferred_element_type=jnp.float32)
    y1_ref[...] = dis[:, None] * xw


def _tc2_body(p_ref, y1_ref, dis_ref, b1_ref, a1_ref, w2_ref, y2_ref):
    dis = dis_ref[...]
    hpre = (dis[:, None] * (p_ref[0, :N] + p_ref[1, :N] + y1_ref[...])
            + b1_ref[...][None, :])
    h = jnp.where(hpre >= 0, hpre, a1_ref[...][None, :] * hpre)
    xw = lax.dot_general(h, w2_ref[...], (((1,), (1,)), ((), ())),
                         preferred_element_type=jnp.float32)
    y2_ref[...] = dis[:, None] * xw


def _tc3_body(p_ref, y2_ref, dis_ref, b2_ref, out_ref):
    out_ref[...] = (dis_ref[...][:, None] * (p_ref[0, :N] + p_ref[1, :N] + y2_ref[...])
                    + b2_ref[...][None, :])


def _vmem_specs(n):
    return [pl.BlockSpec(memory_space=pltpu.VMEM) for _ in range(n)]


_tc1 = pl.pallas_call(
    _tc1_body,
    out_shape=(jax.ShapeDtypeStruct((N,), jnp.float32),
               jax.ShapeDtypeStruct((N, D), jnp.float32)),
    in_specs=_vmem_specs(3),
    out_specs=tuple(_vmem_specs(2)),
)

_tc2 = pl.pallas_call(
    _tc2_body,
    out_shape=jax.ShapeDtypeStruct((N, D), jnp.float32),
    in_specs=_vmem_specs(6),
    out_specs=pl.BlockSpec(memory_space=pltpu.VMEM),
)

_tc3 = pl.pallas_call(
    _tc3_body,
    out_shape=jax.ShapeDtypeStruct((N, D), jnp.float32),
    in_specs=_vmem_specs(4),
    out_specs=pl.BlockSpec(memory_space=pltpu.VMEM),
)


def kernel(x, edge_index, edge_weight, W1, b1, a1, W2, b2):
    row = edge_index[0].astype(jnp.int32).reshape(NW, NB, B)
    col = edge_index[1].astype(jnp.int32).reshape(NW, NB, B)
    colf = edge_index[1].astype(jnp.int32).reshape(NW, ECH)
    ewf = edge_weight.astype(jnp.float32).reshape(NW, ECH)
    ew3 = edge_weight.astype(jnp.float32).reshape(NW, NB, B)
    z1 = jnp.zeros((NP,), jnp.float32)
    znd = jnp.zeros((NP, D), jnp.float32)

    degp = _sc_deg(colf, ewf, z1)
    dis, y1 = _tc1(degp, x, W1)
    p1 = _sc_agg(y1, row, col, ew3, znd)
    y2 = _tc2(p1, y1, dis, b1, a1, W2)
    p2 = _sc_agg(y2, row, col, ew3, znd)
    return _tc3(p2, y2, dis, b2)


# speedup vs baseline: 1.0031x; 1.0031x over previous
import functools

import jax
import jax.numpy as jnp
from jax import lax
from jax.experimental import pallas as pl
from jax.experimental.pallas import tpu as pltpu
from jax.experimental.pallas import tpu_sc as plsc

N = 10000
NP = 10240
E = 320000
D = 128

NC = 2
NS = 16
NW = NC * NS
ECH = E // NW
B = 100
NB = ECH // B
ROWS_S = NP // NS

_mesh = plsc.VectorSubcoreMesh(core_axis_name="c", subcore_axis_name="s")
_sc_params = pltpu.CompilerParams(needs_layout_passes=False)


@functools.partial(
    pl.kernel,
    out_type=jax.ShapeDtypeStruct((NW, NP), jnp.float32),
    mesh=_mesh,
    scratch_types=[
        pltpu.VMEM((NP,), jnp.float32),
        pltpu.VMEM((ECH,), jnp.int32),
        pltpu.VMEM((ECH,), jnp.float32),
    ],
    compiler_params=_sc_params,
)
def _sc_deg(col_hbm, ew_hbm, z_hbm, out_hbm, deg_v, col_v, ew_v):
    c = lax.axis_index("c")
    s = lax.axis_index("s")
    w = c * NS + s
    pltpu.sync_copy(z_hbm, deg_v)
    pltpu.sync_copy(col_hbm.at[w], col_v)
    pltpu.sync_copy(ew_hbm.at[w], ew_v)

    @pl.loop(0, ECH, step=16)
    def _(i):
        plsc.addupdate_scatter(deg_v, [col_v[pl.ds(i, 16)]], ew_v[pl.ds(i, 16)])

    pltpu.sync_copy(deg_v, out_hbm.at[w])


@functools.partial(
    pl.kernel,
    out_type=jax.ShapeDtypeStruct((NC, NP, D), jnp.float32),
    mesh=_mesh,
    scratch_types=[
        pltpu.VMEM((NB, B), jnp.int32),
        pltpu.VMEM((1, B), jnp.int32),
        pltpu.VMEM((1, B), jnp.int32),
        pltpu.VMEM((B,), jnp.float32),
        pltpu.VMEM((B,), jnp.float32),
        pltpu.VMEM((B, D), jnp.float32),
        pltpu.VMEM((B, D), jnp.float32),
        pltpu.SemaphoreType.DMA,
        pltpu.SemaphoreType.DMA,
        pltpu.SemaphoreType.DMA,
        pltpu.SemaphoreType.DMA,
        pltpu.SemaphoreType.DMA,
        pltpu.SemaphoreType.DMA,
        pltpu.SemaphoreType.DMA,
        pltpu.SemaphoreType.DMA,
        pltpu.VMEM_SHARED((NP, D), jnp.float32),
    ],
    compiler_params=_sc_params,
)
def _sc_agg(y_hbm, row_hbm, col_hbm, ew_hbm, z_hbm, out_hbm,
            row_v, colb0, colb1, ewb0, ewb1, rows_v0, rows_v1,
            csem0, csem1, esem0, esem1, gsem0, gsem1, ssem0, ssem1, acc_sh):
    c = lax.axis_index("c")
    s = lax.axis_index("s")
    w = c * NS + s
    rows = (rows_v0, rows_v1)
    colb = (colb0, colb1)
    ewb = (ewb0, ewb1)
    csem = (csem0, csem1)
    esem = (esem0, esem1)
    gsem = (gsem0, gsem1)
    ssem = (ssem0, ssem1)
    pltpu.sync_copy(z_hbm.at[pl.ds(s * ROWS_S, ROWS_S)],
                    acc_sh.at[pl.ds(s * ROWS_S, ROWS_S)])
    pltpu.sync_copy(row_hbm.at[w], row_v)
    plsc.subcore_barrier()

    def scale(b):
        @plsc.parallel_loop(0, B, step=1, unroll=2)
        def _(e):
            b16 = plsc.load_gather(ewb[b], [jnp.full((16,), e, jnp.int32)])
            for k in range(D // 16):
                rows[b][e, pl.ds(k * 16, 16)] = (
                    rows[b][e, pl.ds(k * 16, 16)] * b16)

    def body(j, b, nxt=True, drain=True):
        nb = b ^ 1
        if drain:
            pltpu.make_async_copy(rows[nb], acc_sh.at[colb[nb].at[0]],
                                  ssem[nb]).wait()
        if nxt:
            pltpu.async_copy(y_hbm.at[row_v.at[j + 1]], rows[nb], gsem[nb])
            pltpu.async_copy(col_hbm.at[w, pl.ds(j + 1, 1)], colb[nb],
                             csem[nb])
            pltpu.async_copy(ew_hbm.at[w, j + 1], ewb[nb], esem[nb])
        pltpu.make_async_copy(y_hbm.at[row_v.at[j]], rows[b],
                              gsem[b]).wait()
        pltpu.make_async_copy(ew_hbm.at[w, j], ewb[b], esem[b]).wait()
        scale(b)
        pltpu.make_async_copy(col_hbm.at[w, pl.ds(j, 1)], colb[b],
                              csem[b]).wait()
        pltpu.async_copy(rows[b], acc_sh.at[colb[b].at[0]], ssem[b],
                         add=True)

    pltpu.async_copy(y_hbm.at[row_v.at[0]], rows[0], gsem[0])
    pltpu.async_copy(col_hbm.at[w, pl.ds(0, 1)], colb[0], csem[0])
    pltpu.async_copy(ew_hbm.at[w, 0], ewb[0], esem[0])

    body(0, 0, drain=False)
    body(1, 1)

    @pl.loop(2, NB - 2, step=2)
    def _(j):
        body(j, 0)
        body(j + 1, 1)

    body(NB - 2, 0)
    body(NB - 1, 1, nxt=False)
    pltpu.make_async_copy(rows[1], acc_sh.at[colb[1].at[0]], ssem[1]).wait()

    plsc.subcore_barrier()
    pltpu.sync_copy(acc_sh.at[pl.ds(s * ROWS_S, ROWS_S)],
                    out_hbm.at[c, pl.ds(s * ROWS_S, ROWS_S)])


def _tc1_body(degp_ref, x_ref, w1_ref, dis_ref, y1_ref):
    deg = jnp.sum(degp_ref[:, :N], axis=0) + 1.0
    dis = lax.rsqrt(deg)
    dis_ref[...] = dis
    xw = lax.dot_general(x_ref[...], w1_ref[...], (((1,), (1,)), ((), ())),
                         preferred_element_type=jnp.float32)
    y1_ref[...] = dis[:, None] * xw


def _tc2_body(p_ref, y1_ref, dis_ref, b1_ref, a1_ref, w2_ref, y2_ref):
    dis = dis_ref[...]
    hpre = (dis[:, None] * (p_ref[0, :N] + p_ref[1, :N] + y1_ref[...])
            + b1_ref[...][None, :])
    h = jnp.where(hpre >= 0, hpre, a1_ref[...][None, :] * hpre)
    xw = lax.dot_general(h, w2_ref[...], (((1,), (1,)), ((), ())),
                         preferred_element_type=jnp.float32)
    y2_ref[...] = dis[:, None] * xw


def _tc3_body(p_ref, y2_ref, dis_ref, b2_ref, out_ref):
    out_ref[...] = (dis_ref[...][:, None] * (p_ref[0, :N] + p_ref[1, :N] + y2_ref[...])
                    + b2_ref[...][None, :])


def _vmem_specs(n):
    return [pl.BlockSpec(memory_space=pltpu.VMEM) for _ in range(n)]


_tc1 = pl.pallas_call(
    _tc1_body,
    out_shape=(jax.ShapeDtypeStruct((N,), jnp.float32),
               jax.ShapeDtypeStruct((N, D), jnp.float32)),
    in_specs=_vmem_specs(3),
    out_specs=tuple(_vmem_specs(2)),
)

_tc2 = pl.pallas_call(
    _tc2_body,
    out_shape=jax.ShapeDtypeStruct((N, D), jnp.float32),
    in_specs=_vmem_specs(6),
    out_specs=pl.BlockSpec(memory_space=pltpu.VMEM),
)

_tc3 = pl.pallas_call(
    _tc3_body,
    out_shape=jax.ShapeDtypeStruct((N, D), jnp.float32),
    in_specs=_vmem_specs(4),
    out_specs=pl.BlockSpec(memory_space=pltpu.VMEM),
)


def kernel(x, edge_index, edge_weight, W1, b1, a1, W2, b2):
    row = edge_index[0].astype(jnp.int32).reshape(NW, NB, B)
    col = edge_index[1].astype(jnp.int32).reshape(NW, NB, B)
    colf = edge_index[1].astype(jnp.int32).reshape(NW, ECH)
    ewf = edge_weight.astype(jnp.float32).reshape(NW, ECH)
    ew3 = edge_weight.astype(jnp.float32).reshape(NW, NB, B)
    z1 = jnp.zeros((NP,), jnp.float32)
    znd = jnp.zeros((NP, D), jnp.float32)

    degp = _sc_deg(colf, ewf, z1)
    dis, y1 = _tc1(degp, x, W1)
    p1 = _sc_agg(y1, row, col, ew3, znd)
    y2 = _tc2(p1, y1, dis, b1, a1, W2)
    p2 = _sc_agg(y2, row, col, ew3, znd)
    return _tc3(p2, y2, dis, b2)
